# baseline (speedup 1.0000x reference)
import jax
import jax.numpy as jnp
from jax.experimental import pallas as pl
from jax.experimental.pallas import tpu as pltpu

TILE_V = 2048
HALF_V = TILE_V // 2


def _proj_kernel(x_ref, wa_ref, wb_ref, b_ref, o_ref):
    i = pl.program_id(0)
    x = x_ref[:, :]
    dn = (((1,), (1,)), ((), ()))
    acc_a = jax.lax.dot_general(x, wa_ref[:, :], dimension_numbers=dn,
                                preferred_element_type=jnp.float32)
    acc_b = jax.lax.dot_general(x, wb_ref[:, :], dimension_numbers=dn,
                                preferred_element_type=jnp.float32)
    base = i * TILE_V
    o_ref[:, :HALF_V] = acc_a + b_ref[:, pl.ds(base, HALF_V)]
    o_ref[:, HALF_V:] = acc_b + b_ref[:, pl.ds(base + HALF_V, HALF_V)]


def kernel(x, W, b):
    batch, embed = x.shape
    vocab = W.shape[0]
    num_tiles = pl.cdiv(vocab, TILE_V)
    b2 = b.reshape(1, vocab)
    return pl.pallas_call(
        _proj_kernel,
        grid=(num_tiles,),
        in_specs=[
            pl.BlockSpec((batch, embed), lambda i: (0, 0)),
            pl.BlockSpec((HALF_V, embed), lambda i: (2 * i, 0)),
            pl.BlockSpec((HALF_V, embed), lambda i: (2 * i + 1, 0)),
            pl.BlockSpec((1, num_tiles * TILE_V), lambda i: (0, 0)),
        ],
        out_specs=pl.BlockSpec((batch, TILE_V), lambda i: (0, i)),
        out_shape=jax.ShapeDtypeStruct((batch, vocab), jnp.float32),
        compiler_params=pltpu.CompilerParams(
            dimension_semantics=("arbitrary",),
        ),
    )(x, W, W, b2)


# speedup vs baseline: 1.0101x; 1.0101x over previous
import jax
import jax.numpy as jnp
from jax.experimental import pallas as pl
from jax.experimental.pallas import tpu as pltpu

TILE_V = 2048
TAIL_SIZES = (1152, 512)


def _chunk_plan(vocab):
    aligned_end = (vocab // 128) * 128
    rem = vocab - aligned_end
    tail = sum(TAIL_SIZES)
    n_full = (aligned_end - tail) // TILE_V
    left = aligned_end - tail - n_full * TILE_V
    sizes = [TILE_V] * n_full
    if left:
        sizes.append(left)
    sizes.extend(TAIL_SIZES)
    offs, o = [], 0
    for s in sizes:
        offs.append(o)
        o += s
    assert o == aligned_end
    return tuple(zip(offs, sizes)), aligned_end, rem


def kernel(x, W, b):
    batch, embed = x.shape
    vocab = W.shape[0]
    plan, rem_off, rem = _chunk_plan(vocab)
    n = len(plan)
    b2 = b.reshape(1, vocab)
    b_tail = jax.lax.slice(b, (rem_off,), (vocab,)).reshape(1, rem)

    def body(x_ref, w_hbm, b_ref, bt_ref, o_hbm,
             wbuf, obuf, wtbuf, otbuf, wsem, osem, tsem):
        def w_copy(c):
            off, size = plan[c]
            slot = c % 2
            return pltpu.make_async_copy(
                w_hbm.at[pl.ds(off, size), :],
                wbuf.at[slot, pl.ds(0, size), :],
                wsem.at[slot],
            )

        def o_copy(c):
            off, size = plan[c]
            slot = c % 2
            return pltpu.make_async_copy(
                obuf.at[slot, :, pl.ds(0, size)],
                o_hbm.at[:, pl.ds(off, size)],
                osem.at[slot],
            )

        wt_copy = pltpu.make_async_copy(
            w_hbm.at[pl.ds(rem_off, rem), :], wtbuf, tsem)
        ot_copy = pltpu.make_async_copy(
            otbuf, o_hbm.at[:, pl.ds(rem_off, rem)], tsem)

        w_copy(0).start()
        w_copy(1).start()
        x = x_ref[:, :]
        dn = (((1,), (1,)), ((), ()))
        for c in range(n):
            off, size = plan[c]
            slot = c % 2
            w_copy(c).wait()
            if c >= 2:
                o_copy(c - 2).wait()
            acc = jax.lax.dot_general(
                x, wbuf[slot, pl.ds(0, size), :],
                dimension_numbers=dn, preferred_element_type=jnp.float32,
            )
            obuf[slot, :, pl.ds(0, size)] = acc + b_ref[:, pl.ds(off, size)]
            if c + 2 < n:
                w_copy(c + 2).start()
            elif c + 2 == n:
                wt_copy.start()
            o_copy(c).start()
        wt_copy.wait()
        acc = jax.lax.dot_general(
            x, wtbuf[:, :],
            dimension_numbers=dn, preferred_element_type=jnp.float32,
        )
        otbuf[:, :] = acc + bt_ref[:, :]
        ot_copy.start()
        o_copy(n - 2).wait()
        o_copy(n - 1).wait()
        ot_copy.wait()

    return pl.pallas_call(
        body,
        in_specs=[
            pl.BlockSpec((batch, embed), lambda: (0, 0)),
            pl.BlockSpec(memory_space=pltpu.MemorySpace.HBM),
            pl.BlockSpec((1, vocab), lambda: (0, 0)),
            pl.BlockSpec((1, rem), lambda: (0, 0)),
        ],
        out_specs=pl.BlockSpec(memory_space=pltpu.MemorySpace.HBM),
        out_shape=jax.ShapeDtypeStruct((batch, vocab), jnp.float32),
        scratch_shapes=[
            pltpu.VMEM((2, TILE_V, embed), jnp.float32),
            pltpu.VMEM((2, batch, TILE_V), jnp.float32),
            pltpu.VMEM((rem, embed), jnp.float32),
            pltpu.VMEM((batch, rem), jnp.float32),
            pltpu.SemaphoreType.DMA((2,)),
            pltpu.SemaphoreType.DMA((2,)),
            pltpu.SemaphoreType.DMA,
        ],
    )(x, W, b2, b_tail)
